# baseline (speedup 1.0000x reference)
import functools

import jax
import jax.numpy as jnp
from jax import lax
from jax.experimental import pallas as pl
from jax.experimental.pallas import tpu as pltpu
from jax.experimental.pallas import tpu_sc as plsc

NC = 2
NS = 16
LANES = 16
K = 80
RB = 4
IB = 5


def _sc_degree(NPAD, RPT, ITERS):
    mesh = plsc.VectorSubcoreMesh(core_axis_name="c", subcore_axis_name="s")

    @functools.partial(
        pl.kernel,
        out_type=jax.ShapeDtypeStruct((NC, NS, RPT), jnp.float32),
        mesh=mesh,
        scratch_types=[
            pltpu.VMEM((ITERS, 1, K), jnp.int32),
            pltpu.VMEM((K,), jnp.float32),
            pltpu.VMEM((RPT,), jnp.float32),
            pltpu.VMEM_SHARED((NPAD,), jnp.float32),
            pltpu.SemaphoreType.DMA,
        ],
    )
    def deg_kernel(dst_hbm, out_hbm, idxb, ones_v, zvec_v, deg_sh, ssem):
        c = lax.axis_index("c")
        s = lax.axis_index("s")
        wid = s * NC + c

        pltpu.sync_copy(dst_hbm.at[wid], idxb)

        def zfill(i, carry):
            zvec_v[pl.ds(i * LANES, LANES)] = jnp.zeros((LANES,), jnp.float32)
            return carry

        lax.fori_loop(0, RPT // LANES, zfill, 0)
        pltpu.sync_copy(zvec_v, deg_sh.at[pl.ds(s * RPT, RPT)])
        for j in range(K // LANES):
            ones_v[pl.ds(j * LANES, LANES)] = jnp.ones((LANES,), jnp.float32)
        plsc.subcore_barrier()

        def fire(i, carry):
            pltpu.async_copy(ones_v, deg_sh.at[idxb.at[i, 0]], ssem, add=True)
            return carry

        lax.fori_loop(0, ITERS, fire, 0)

        def drain(i, carry):
            pltpu.make_async_copy(ones_v, deg_sh.at[idxb.at[i, 0]], ssem).wait()
            return carry

        lax.fori_loop(0, ITERS, drain, 0)
        plsc.subcore_barrier()
        pltpu.sync_copy(deg_sh.at[pl.ds(s * RPT, RPT)], out_hbm.at[c, s])

    return deg_kernel


def _sc_aggregate(NPAD, RPT, E_PER, ITERS, D):
    mesh = plsc.VectorSubcoreMesh(core_axis_name="c", subcore_axis_name="s")

    @functools.partial(
        pl.kernel,
        out_type=jax.ShapeDtypeStruct((NC, NS, RPT, D), jnp.float32),
        mesh=mesh,
        scratch_types=[
            pltpu.VMEM((IB, 2, K), jnp.int32),
            pltpu.VMEM((RB, K, D), jnp.float32),
            pltpu.VMEM_SHARED((NPAD, D), jnp.float32),
            pltpu.SemaphoreType.DMA((IB,)),
            pltpu.SemaphoreType.DMA((RB,)),
            pltpu.SemaphoreType.DMA((RB,)),
        ],
    )
    def agg_kernel(h_hbm, src_hbm, dst_hbm, out_hbm, ibuf, rows_v, accum_sh, isem, gsem, ssem):
        c = lax.axis_index("c")
        s = lax.axis_index("s")
        wid = s * NC + c

        def zfill(r, carry):
            for j in range(D // LANES):
                rows_v[0, r, pl.ds(j * LANES, LANES)] = jnp.zeros((LANES,), jnp.float32)
            return carry

        lax.fori_loop(0, K, zfill, 0)

        def zcopy(k, carry):
            pltpu.sync_copy(rows_v.at[0], accum_sh.at[pl.ds(s * RPT + k * K, K)])
            return carry

        lax.fori_loop(0, RPT // K, zcopy, 0)
        plsc.subcore_barrier()

        ebase = wid * E_PER

        def icopy(chunk, slot, sem_slot):
            pltpu.async_copy(src_hbm.at[pl.ds(ebase + chunk * K, K)],
                             ibuf.at[slot, 0], isem.at[sem_slot])
            pltpu.async_copy(dst_hbm.at[pl.ds(ebase + chunk * K, K)],
                             ibuf.at[slot, 1], isem.at[sem_slot])

        def iwait(chunk, slot, sem_slot):
            pltpu.make_async_copy(src_hbm.at[pl.ds(ebase + chunk * K, K)],
                                  ibuf.at[slot, 0], isem.at[sem_slot]).wait()
            pltpu.make_async_copy(dst_hbm.at[pl.ds(ebase + chunk * K, K)],
                                  ibuf.at[slot, 1], isem.at[sem_slot]).wait()

        icopy(0, 0, 0)
        iwait(0, 0, 0)
        icopy(1, 1, 1)
        iwait(1, 1, 1)
        pltpu.async_copy(h_hbm.at[ibuf.at[0, 0]], rows_v.at[0], gsem.at[0])
        pltpu.async_copy(h_hbm.at[ibuf.at[1, 0]], rows_v.at[1], gsem.at[1])
        icopy(2, 2, 2)

        def step(i, carry):
            p = lax.rem(i, RB)
            g2 = lax.rem(i + 2, RB)
            pi = lax.rem(i, IB)
            gi = lax.rem(i + 2, IB)
            ni = lax.rem(i + 3, IB)

            @pl.when(i + 2 < ITERS)
            def _():
                iwait(i + 2, gi, gi)

                @pl.when(i >= 2)
                def _():
                    pltpu.make_async_copy(rows_v.at[g2],
                                          accum_sh.at[ibuf.at[gi, 1]],
                                          ssem.at[g2]).wait()

                pltpu.async_copy(h_hbm.at[ibuf.at[gi, 0]], rows_v.at[g2],
                                 gsem.at[g2])

            pltpu.make_async_copy(h_hbm.at[ibuf.at[pi, 0]], rows_v.at[p],
                                  gsem.at[p]).wait()
            pltpu.async_copy(rows_v.at[p], accum_sh.at[ibuf.at[pi, 1]],
                             ssem.at[p], add=True)

            @pl.when(i + 3 < ITERS)
            def _():
                icopy(i + 3, ni, ni)

            return carry

        lax.fori_loop(0, ITERS, step, 0)

        def sdrain(j, carry):
            r = lax.rem(j, RB)
            pltpu.make_async_copy(rows_v.at[r], accum_sh.at[ibuf.at[0, 1]],
                                  ssem.at[r]).wait()
            return carry

        lax.fori_loop(0, RB, sdrain, 0)
        plsc.subcore_barrier()
        pltpu.sync_copy(accum_sh.at[pl.ds(s * RPT, RPT)], out_hbm.at[c, s])

    return agg_kernel


def _norm_from(dp0, dp1):
    deg = dp0 + dp1
    return jnp.where(deg > 0, lax.rsqrt(jnp.maximum(deg, 1.0)), 0.0)


def _tc_linear(N, D, NPAD, Bn):
    def body(x_ref, w_ref, b_ref, dp_ref, out_ref):
        i = pl.program_id(0)
        norm = _norm_from(dp_ref[0, pl.ds(i * Bn, Bn)], dp_ref[1, pl.ds(i * Bn, Bn)])
        h = lax.dot_general(x_ref[...], w_ref[...], (((1,), (1,)), ((), ())),
                            preferred_element_type=jnp.float32)
        out_ref[...] = (h + b_ref[...]) * norm[:, None]

    return pl.pallas_call(
        body,
        grid=(NPAD // Bn,),
        in_specs=[
            pl.BlockSpec((Bn, D), lambda i: (i, 0)),
            pl.BlockSpec((D, D), lambda i: (0, 0)),
            pl.BlockSpec((1, D), lambda i: (0, 0)),
            pl.BlockSpec((NC, NPAD), lambda i: (0, 0)),
        ],
        out_specs=pl.BlockSpec((Bn, D), lambda i: (i, 0)),
        out_shape=jax.ShapeDtypeStruct((N, D), jnp.float32),
    )


def _tc_finish(N, D, NPAD, Bn):
    def body(ap_ref, dp_ref, out_ref):
        i = pl.program_id(0)
        norm = _norm_from(dp_ref[0, pl.ds(i * Bn, Bn)], dp_ref[1, pl.ds(i * Bn, Bn)])
        a = ap_ref[0] + ap_ref[1]
        out_ref[...] = jnp.maximum(a * norm[:, None], 0.0)

    return pl.pallas_call(
        body,
        grid=(NPAD // Bn,),
        in_specs=[
            pl.BlockSpec((NC, Bn, D), lambda i: (0, i, 0)),
            pl.BlockSpec((NC, NPAD), lambda i: (0, 0)),
        ],
        out_specs=pl.BlockSpec((Bn, D), lambda i: (i, 0)),
        out_shape=jax.ShapeDtypeStruct((N, D), jnp.float32),
    )


def kernel(features, edge_index, W, b):
    N, D = features.shape
    E = edge_index.shape[1]
    NW = NC * NS
    E_PER = E // NW
    ITERS = E_PER // K
    NPAD = ((N + NS * LANES - 1) // (NS * LANES)) * (NS * LANES)
    RPT = NPAD // NS
    Bn = 1024

    edge = edge_index.astype(jnp.int32)
    src1 = edge[0]
    dst1 = edge[1]
    dst4 = dst1.reshape(NW, ITERS, 1, K)

    degp = _sc_degree(NPAD, RPT, ITERS)(dst4)
    dp = degp.reshape(NC, NPAD)
    h = _tc_linear(N, D, NPAD, Bn)(features, W, b.reshape(1, D), dp)
    accp = _sc_aggregate(NPAD, RPT, E_PER, ITERS, D)(h, src1, dst1)
    ap = accp.reshape(NC, NPAD, D)
    out = _tc_finish(N, D, NPAD, Bn)(ap, dp)
    return out


# speedup vs baseline: 1.0204x; 1.0204x over previous
import functools

import jax
import jax.numpy as jnp
from jax import lax
from jax.experimental import pallas as pl
from jax.experimental.pallas import tpu as pltpu
from jax.experimental.pallas import tpu_sc as plsc

NC = 2
NS = 16
LANES = 16
K = 80
RB = 4
IB = 5


def _sc_degree(NPAD, RPT, E_PER, ITERS):
    mesh = plsc.VectorSubcoreMesh(core_axis_name="c", subcore_axis_name="s")

    @functools.partial(
        pl.kernel,
        out_type=jax.ShapeDtypeStruct((NC, NS, RPT), jnp.float32),
        mesh=mesh,
        scratch_types=[
            pltpu.VMEM((E_PER,), jnp.int32),
            pltpu.VMEM((ITERS, 1, K), jnp.int32),
            pltpu.VMEM((K,), jnp.float32),
            pltpu.VMEM((RPT,), jnp.float32),
            pltpu.VMEM_SHARED((NPAD,), jnp.float32),
            pltpu.SemaphoreType.DMA,
        ],
    )
    def deg_kernel(pk_hbm, out_hbm, pbuf, idxb, ones_v, zvec_v, deg_sh, ssem):
        c = lax.axis_index("c")
        s = lax.axis_index("s")
        wid = s * NC + c

        pltpu.sync_copy(pk_hbm.at[pl.ds(wid * E_PER, E_PER)], pbuf)

        def decode(m, carry):
            i = m // (K // LANES)
            j = m % (K // LANES)
            v = pbuf[pl.ds(m * LANES, LANES)]
            idxb[i, 0, pl.ds(j * LANES, LANES)] = lax.shift_right_logical(v, 14)
            return carry

        lax.fori_loop(0, ITERS * (K // LANES), decode, 0)

        def zfill(i, carry):
            zvec_v[pl.ds(i * LANES, LANES)] = jnp.zeros((LANES,), jnp.float32)
            return carry

        lax.fori_loop(0, RPT // LANES, zfill, 0)
        pltpu.sync_copy(zvec_v, deg_sh.at[pl.ds(s * RPT, RPT)])
        for j in range(K // LANES):
            ones_v[pl.ds(j * LANES, LANES)] = jnp.ones((LANES,), jnp.float32)
        plsc.subcore_barrier()

        def fire(i, carry):
            pltpu.async_copy(ones_v, deg_sh.at[idxb.at[i, 0]], ssem, add=True)
            return carry

        lax.fori_loop(0, ITERS, fire, 0)

        def drain(i, carry):
            pltpu.make_async_copy(ones_v, deg_sh.at[idxb.at[i, 0]], ssem).wait()
            return carry

        lax.fori_loop(0, ITERS, drain, 0)
        plsc.subcore_barrier()
        pltpu.sync_copy(deg_sh.at[pl.ds(s * RPT, RPT)], out_hbm.at[c, s])

    return deg_kernel


def _sc_aggregate(NPAD, RPT, E_PER, ITERS, D):
    mesh = plsc.VectorSubcoreMesh(core_axis_name="c", subcore_axis_name="s")

    @functools.partial(
        pl.kernel,
        out_type=jax.ShapeDtypeStruct((NC, NS, RPT, D), jnp.float32),
        mesh=mesh,
        scratch_types=[
            pltpu.VMEM((IB, K), jnp.int32),
            pltpu.VMEM((IB, 2, K), jnp.int32),
            pltpu.VMEM((RB, K, D), jnp.float32),
            pltpu.VMEM_SHARED((NPAD, D), jnp.float32),
            pltpu.SemaphoreType.DMA((IB,)),
            pltpu.SemaphoreType.DMA((RB,)),
            pltpu.SemaphoreType.DMA((RB,)),
        ],
    )
    def agg_kernel(h_hbm, pk_hbm, out_hbm, pbuf, ibuf, rows_v, accum_sh, isem, gsem, ssem):
        c = lax.axis_index("c")
        s = lax.axis_index("s")
        wid = s * NC + c

        def zfill(r, carry):
            for j in range(D // LANES):
                rows_v[0, r, pl.ds(j * LANES, LANES)] = jnp.zeros((LANES,), jnp.float32)
            return carry

        lax.fori_loop(0, K, zfill, 0)

        def zcopy(k, carry):
            pltpu.sync_copy(rows_v.at[0], accum_sh.at[pl.ds(s * RPT + k * K, K)])
            return carry

        lax.fori_loop(0, RPT // K, zcopy, 0)
        plsc.subcore_barrier()

        ebase = wid * E_PER

        def icopy(chunk, slot, sem_slot):
            pltpu.async_copy(pk_hbm.at[pl.ds(ebase + chunk * K, K)],
                             pbuf.at[slot], isem.at[sem_slot])

        def iwait(chunk, slot, sem_slot):
            pltpu.make_async_copy(pk_hbm.at[pl.ds(ebase + chunk * K, K)],
                                  pbuf.at[slot], isem.at[sem_slot]).wait()
            for j in range(K // LANES):
                v = pbuf[slot, pl.ds(j * LANES, LANES)]
                ibuf[slot, 0, pl.ds(j * LANES, LANES)] = lax.bitwise_and(
                    v, jnp.full((LANES,), 16383, jnp.int32))
                ibuf[slot, 1, pl.ds(j * LANES, LANES)] = lax.shift_right_logical(
                    v, 14)

        icopy(0, 0, 0)
        iwait(0, 0, 0)
        icopy(1, 1, 1)
        iwait(1, 1, 1)
        pltpu.async_copy(h_hbm.at[ibuf.at[0, 0]], rows_v.at[0], gsem.at[0])
        pltpu.async_copy(h_hbm.at[ibuf.at[1, 0]], rows_v.at[1], gsem.at[1])
        icopy(2, 2, 2)

        def step(i, carry):
            p = lax.rem(i, RB)
            g2 = lax.rem(i + 2, RB)
            pi = lax.rem(i, IB)
            gi = lax.rem(i + 2, IB)
            ni = lax.rem(i + 3, IB)

            @pl.when(i + 2 < ITERS)
            def _():
                iwait(i + 2, gi, gi)

                @pl.when(i >= 2)
                def _():
                    pltpu.make_async_copy(rows_v.at[g2],
                                          accum_sh.at[ibuf.at[gi, 1]],
                                          ssem.at[g2]).wait()

                pltpu.async_copy(h_hbm.at[ibuf.at[gi, 0]], rows_v.at[g2],
                                 gsem.at[g2])

            pltpu.make_async_copy(h_hbm.at[ibuf.at[pi, 0]], rows_v.at[p],
                                  gsem.at[p]).wait()
            pltpu.async_copy(rows_v.at[p], accum_sh.at[ibuf.at[pi, 1]],
                             ssem.at[p], add=True)

            @pl.when(i + 3 < ITERS)
            def _():
                icopy(i + 3, ni, ni)

            return carry

        lax.fori_loop(0, ITERS, step, 0)

        def sdrain(j, carry):
            r = lax.rem(j, RB)
            pltpu.make_async_copy(rows_v.at[r], accum_sh.at[ibuf.at[0, 1]],
                                  ssem.at[r]).wait()
            return carry

        lax.fori_loop(0, RB, sdrain, 0)
        plsc.subcore_barrier()
        pltpu.sync_copy(accum_sh.at[pl.ds(s * RPT, RPT)], out_hbm.at[c, s])

    return agg_kernel


def _norm_from(dp0, dp1):
    deg = dp0 + dp1
    return jnp.where(deg > 0, lax.rsqrt(jnp.maximum(deg, 1.0)), 0.0)


def _tc_linear(N, D, NPAD, Bn):
    def body(x_ref, w_ref, b_ref, dp_ref, out_ref):
        i = pl.program_id(0)
        norm = _norm_from(dp_ref[0, pl.ds(i * Bn, Bn)], dp_ref[1, pl.ds(i * Bn, Bn)])
        h = lax.dot_general(x_ref[...], w_ref[...], (((1,), (1,)), ((), ())),
                            preferred_element_type=jnp.float32)
        out_ref[...] = (h + b_ref[...]) * norm[:, None]

    return pl.pallas_call(
        body,
        grid=(NPAD // Bn,),
        in_specs=[
            pl.BlockSpec((Bn, D), lambda i: (i, 0)),
            pl.BlockSpec((D, D), lambda i: (0, 0)),
            pl.BlockSpec((1, D), lambda i: (0, 0)),
            pl.BlockSpec((NC, NPAD), lambda i: (0, 0)),
        ],
        out_specs=pl.BlockSpec((Bn, D), lambda i: (i, 0)),
        out_shape=jax.ShapeDtypeStruct((N, D), jnp.float32),
    )


def _tc_finish(N, D, NPAD, Bn):
    def body(ap_ref, dp_ref, out_ref):
        i = pl.program_id(0)
        norm = _norm_from(dp_ref[0, pl.ds(i * Bn, Bn)], dp_ref[1, pl.ds(i * Bn, Bn)])
        a = ap_ref[0] + ap_ref[1]
        out_ref[...] = jnp.maximum(a * norm[:, None], 0.0)

    return pl.pallas_call(
        body,
        grid=(NPAD // Bn,),
        in_specs=[
            pl.BlockSpec((NC, Bn, D), lambda i: (0, i, 0)),
            pl.BlockSpec((NC, NPAD), lambda i: (0, 0)),
        ],
        out_specs=pl.BlockSpec((Bn, D), lambda i: (i, 0)),
        out_shape=jax.ShapeDtypeStruct((N, D), jnp.float32),
    )


def kernel(features, edge_index, W, b):
    N, D = features.shape
    E = edge_index.shape[1]
    NW = NC * NS
    E_PER = E // NW
    ITERS = E_PER // K
    NPAD = ((N + NS * LANES - 1) // (NS * LANES)) * (NS * LANES)
    RPT = NPAD // NS
    Bn = 1024

    edge = edge_index.astype(jnp.int32)
    packed = edge[0] + edge[1] * 16384

    degp = _sc_degree(NPAD, RPT, E_PER, ITERS)(packed)
    dp = degp.reshape(NC, NPAD)
    h = _tc_linear(N, D, NPAD, Bn)(features, W, b.reshape(1, D), dp)
    accp = _sc_aggregate(NPAD, RPT, E_PER, ITERS, D)(h, packed)
    ap = accp.reshape(NC, NPAD, D)
    out = _tc_finish(N, D, NPAD, Bn)(ap, dp)
    return out


# speedup vs baseline: 1.0831x; 1.0614x over previous
import functools

import jax
import jax.numpy as jnp
from jax import lax
from jax.experimental import pallas as pl
from jax.experimental.pallas import tpu as pltpu
from jax.experimental.pallas import tpu_sc as plsc

NC = 2
NS = 16
LANES = 16
K = 128
RB = 2
IB = 4
MAXCH = 79


def _tile_chunks(wid, total_chunks):
    nw = NC * NS
    base = total_chunks // nw
    rem = total_chunks - base * nw
    nch = base + jnp.where(wid < rem, 1, 0)
    c0 = wid * base + jnp.minimum(wid, rem)
    return c0, nch


def _sc_degree(NPAD, RPT, TCH):
    mesh = plsc.VectorSubcoreMesh(core_axis_name="c", subcore_axis_name="s")

    @functools.partial(
        pl.kernel,
        out_type=jax.ShapeDtypeStruct((NC, NPAD), jnp.float32),
        mesh=mesh,
        scratch_types=[
            pltpu.VMEM((MAXCH, 2, K), jnp.int32),
            pltpu.VMEM((K,), jnp.float32),
            pltpu.VMEM((RPT,), jnp.float32),
            pltpu.VMEM_SHARED((NPAD,), jnp.float32),
            pltpu.SemaphoreType.DMA,
            pltpu.SemaphoreType.DMA,
        ],
    )
    def deg_kernel(e_hbm, out_hbm, idxb, ones_v, zvec_v, deg_sh, lsem, ssem):
        c = lax.axis_index("c")
        s = lax.axis_index("s")
        wid = s * NC + c
        c0, nch = _tile_chunks(wid, TCH)

        def stage(i, carry):
            pltpu.async_copy(e_hbm.at[:, pl.ds((c0 + i) * K, K)], idxb.at[i],
                             lsem)
            return carry

        lax.fori_loop(0, nch, stage, 0)

        def zfill(i, carry):
            zvec_v[pl.ds(i * LANES, LANES)] = jnp.zeros((LANES,), jnp.float32)
            return carry

        lax.fori_loop(0, RPT // LANES, zfill, 0)
        pltpu.sync_copy(zvec_v, deg_sh.at[pl.ds(s * RPT, RPT)])
        for j in range(K // LANES):
            ones_v[pl.ds(j * LANES, LANES)] = jnp.ones((LANES,), jnp.float32)

        def sdrain(i, carry):
            pltpu.make_async_copy(e_hbm.at[:, pl.ds((c0 + i) * K, K)],
                                  idxb.at[i], lsem).wait()
            return carry

        lax.fori_loop(0, nch, sdrain, 0)
        plsc.subcore_barrier()

        def fire(i, carry):
            pltpu.async_copy(ones_v, deg_sh.at[idxb.at[i, 1]], ssem, add=True)
            return carry

        lax.fori_loop(0, nch, fire, 0)

        def drain(i, carry):
            pltpu.make_async_copy(ones_v, deg_sh.at[idxb.at[i, 1]], ssem).wait()
            return carry

        lax.fori_loop(0, nch, drain, 0)
        plsc.subcore_barrier()
        pltpu.sync_copy(deg_sh.at[pl.ds(s * RPT, RPT)],
                        out_hbm.at[c, pl.ds(s * RPT, RPT)])

    return deg_kernel


def _sc_aggregate(NPAD, RPT, TCH, D):
    mesh = plsc.VectorSubcoreMesh(core_axis_name="c", subcore_axis_name="s")

    @functools.partial(
        pl.kernel,
        out_type=jax.ShapeDtypeStruct((NC, NS, RPT, D), jnp.float32),
        mesh=mesh,
        scratch_types=[
            pltpu.VMEM((IB, 2, K), jnp.int32),
            pltpu.VMEM((RB, K, D), jnp.float32),
            pltpu.VMEM_SHARED((NPAD, D), jnp.float32),
            pltpu.SemaphoreType.DMA((IB,)),
            pltpu.SemaphoreType.DMA((RB,)),
            pltpu.SemaphoreType.DMA((RB,)),
        ],
    )
    def agg_kernel(h_hbm, e_hbm, out_hbm, ibuf, rows_v, accum_sh, isem, gsem, ssem):
        c = lax.axis_index("c")
        s = lax.axis_index("s")
        wid = s * NC + c
        c0, nch = _tile_chunks(wid, TCH)

        def icopy(chunk, slot):
            pltpu.async_copy(e_hbm.at[:, pl.ds((c0 + chunk) * K, K)],
                             ibuf.at[slot], isem.at[slot])

        def iwait(chunk, slot):
            pltpu.make_async_copy(e_hbm.at[:, pl.ds((c0 + chunk) * K, K)],
                                  ibuf.at[slot], isem.at[slot]).wait()

        def zfill(r, carry):
            for j in range(D // LANES):
                rows_v[0, r, pl.ds(j * LANES, LANES)] = jnp.zeros((LANES,), jnp.float32)
            return carry

        lax.fori_loop(0, K, zfill, 0)

        def zcopy(k, carry):
            pltpu.sync_copy(rows_v.at[0], accum_sh.at[pl.ds(s * RPT + k * K, K)])
            return carry

        lax.fori_loop(0, RPT // K, zcopy, 0)
        plsc.subcore_barrier()

        icopy(0, 0)
        iwait(0, 0)
        pltpu.async_copy(h_hbm.at[ibuf.at[0, 0]], rows_v.at[0], gsem.at[0])
        icopy(1, 1)

        def step(i, carry):
            p = lax.rem(i, RB)
            q = lax.rem(i + 1, RB)
            pi = lax.rem(i, IB)
            qi = lax.rem(i + 1, IB)
            ni = lax.rem(i + 2, IB)

            @pl.when(i + 1 < nch)
            def _():
                iwait(i + 1, qi)

                @pl.when(i >= 1)
                def _():
                    pltpu.make_async_copy(rows_v.at[q],
                                          accum_sh.at[ibuf.at[qi, 1]],
                                          ssem.at[q]).wait()

                pltpu.async_copy(h_hbm.at[ibuf.at[qi, 0]], rows_v.at[q],
                                 gsem.at[q])

            pltpu.make_async_copy(h_hbm.at[ibuf.at[pi, 0]], rows_v.at[p],
                                  gsem.at[p]).wait()
            pltpu.async_copy(rows_v.at[p], accum_sh.at[ibuf.at[pi, 1]],
                             ssem.at[p], add=True)

            @pl.when(i + 2 < nch)
            def _():
                icopy(i + 2, ni)

            return carry

        lax.fori_loop(0, nch, step, 0)

        def fdrain(j, carry):
            r = lax.rem(j, RB)
            pltpu.make_async_copy(rows_v.at[r], accum_sh.at[ibuf.at[0, 1]],
                                  ssem.at[r]).wait()
            return carry

        lax.fori_loop(0, RB, fdrain, 0)
        plsc.subcore_barrier()
        pltpu.sync_copy(accum_sh.at[pl.ds(s * RPT, RPT)], out_hbm.at[c, s])

    return agg_kernel


def _norm_from(dp0, dp1):
    deg = dp0 + dp1
    return jnp.where(deg > 0, lax.rsqrt(jnp.maximum(deg, 1.0)), 0.0)


def _tc_linear(N, D, NPAD, Bn):
    def body(x_ref, w_ref, b_ref, dp_ref, out_ref):
        i = pl.program_id(0)
        norm = _norm_from(dp_ref[0, pl.ds(i * Bn, Bn)], dp_ref[1, pl.ds(i * Bn, Bn)])
        h = lax.dot_general(x_ref[...], w_ref[...], (((1,), (1,)), ((), ())),
                            preferred_element_type=jnp.float32)
        out_ref[...] = (h + b_ref[...]) * norm[:, None]

    return pl.pallas_call(
        body,
        grid=(NPAD // Bn,),
        in_specs=[
            pl.BlockSpec((Bn, D), lambda i: (i, 0)),
            pl.BlockSpec((D, D), lambda i: (0, 0)),
            pl.BlockSpec((1, D), lambda i: (0, 0)),
            pl.BlockSpec((NC, NPAD), lambda i: (0, 0)),
        ],
        out_specs=pl.BlockSpec((Bn, D), lambda i: (i, 0)),
        out_shape=jax.ShapeDtypeStruct((N, D), jnp.float32),
    )


def _tc_finish(N, D, NPAD, Bn):
    def body(ap_ref, dp_ref, out_ref):
        i = pl.program_id(0)
        norm = _norm_from(dp_ref[0, pl.ds(i * Bn, Bn)], dp_ref[1, pl.ds(i * Bn, Bn)])
        a = ap_ref[0] + ap_ref[1]
        out_ref[...] = jnp.maximum(a * norm[:, None], 0.0)

    return pl.pallas_call(
        body,
        grid=(NPAD // Bn,),
        in_specs=[
            pl.BlockSpec((NC, Bn, D), lambda i: (0, i, 0)),
            pl.BlockSpec((NC, NPAD), lambda i: (0, 0)),
        ],
        out_specs=pl.BlockSpec((Bn, D), lambda i: (i, 0)),
        out_shape=jax.ShapeDtypeStruct((N, D), jnp.float32),
    )


def kernel(features, edge_index, W, b):
    N, D = features.shape
    E = edge_index.shape[1]
    TCH = E // K
    NPAD = ((N + NS * LANES - 1) // (NS * LANES)) * (NS * LANES)
    RPT = NPAD // NS
    Bn = 1024

    edge = edge_index.astype(jnp.int32)

    dp = _sc_degree(NPAD, RPT, TCH)(edge)
    h = _tc_linear(N, D, NPAD, Bn)(features, W, b.reshape(1, D), dp)
    ap = _sc_aggregate(NPAD, RPT, TCH, D)(h, edge).reshape(NC, NPAD, D)
    out = _tc_finish(N, D, NPAD, Bn)(ap, dp)
    return out
